# Initial kernel scaffold; baseline (speedup 1.0000x reference)
import jax
import jax.numpy as jnp
from jax.experimental import pallas as pl


def kernel(z, W, supports_bank, labels_bank, ent_bank, scores_bank):
    raise NotImplementedError("write your pallas kernel here")


# speedup vs baseline: 4.2470x; 4.2470x over previous
import jax
import jax.numpy as jnp
from jax.experimental import pallas as pl
from jax.experimental.pallas import tpu as pltpu

_B, _D, _C, _NBANK, _FK, _LAM = 256, 2048, 200, 20000, 100, 0.1
_N = _NBANK + _B
_NP = 20480
_CP = 256
_TS = 512
_TK = 2048
_NEG = -1e30


def _head_k(z_ref, w_ref, p_ref, sc_ref, ent_ref, cls_ref, zn_ref):
    z = z_ref[...]
    w = w_ref[...]
    p = jax.lax.dot_general(z, w, (((1,), (1,)), ((), ())),
                            preferred_element_type=jnp.float32)
    lane = jax.lax.broadcasted_iota(jnp.int32, (_B, _CP), 1)
    maskc = lane < _C
    logits = jnp.where(maskc, p, _NEG)
    m = jnp.max(logits, axis=1, keepdims=True)
    e = jnp.exp(logits - m)
    s = jnp.sum(e, axis=1, keepdims=True)
    sc = e / s
    lsm = logits - m - jnp.log(s)
    ent = -jnp.sum(jnp.where(maskc, sc * lsm, 0.0), axis=1, keepdims=True)
    pos = jnp.min(jnp.where(logits == m, lane, _CP), axis=1, keepdims=True)
    n1 = jnp.sum(jnp.abs(z), axis=1, keepdims=True)
    p_ref[...] = p
    sc_ref[...] = sc
    ent_ref[...] = ent
    cls_ref[...] = pos.astype(jnp.float32)
    zn_ref[...] = z / jnp.maximum(n1, 1e-12)


def _cls_k(l_ref, o_ref):
    lab = l_ref[...]
    lane = jax.lax.broadcasted_iota(jnp.int32, lab.shape, 1).astype(jnp.float32)
    o_ref[...] = jnp.sum(lab * lane, axis=1, keepdims=True)


def _rank_k(ec_ref, cc_ref, er_ref, cr_ref, v_ref):
    i = pl.program_id(0)
    ej = ec_ref[...]
    cj = cc_ref[...]
    idxj = jax.lax.broadcasted_iota(jnp.int32, (_TS, 1), 0) + i * _TS
    cnt = jnp.zeros((_TS, 1), jnp.float32)
    for k0 in range(0, _NP, _TK):
        ek = er_ref[:, k0:k0 + _TK]
        ck = cr_ref[:, k0:k0 + _TK]
        idxk = jax.lax.broadcasted_iota(jnp.int32, (1, _TK), 1) + k0
        same = ck == cj
        less = (ek < ej) | ((ek == ej) & (idxk < idxj))
        cnt = cnt + jnp.sum(jnp.where(same & less, 1.0, 0.0),
                            axis=1, keepdims=True)
    v = (cnt < _FK) & (idxj < _N)
    v_ref[...] = v.astype(jnp.float32)


def _big_k(s_ref, v_ref, c_ref, zn_ref, sim_ref, wt_ref):
    i = pl.program_id(0)
    S = s_ref[...]
    n2 = jnp.sum(S * S, axis=1, keepdims=True)
    s2 = S * (1.0 / jnp.maximum(jnp.sqrt(n2), 1e-12))
    n1 = jnp.sum(jnp.abs(s2), axis=1, keepdims=True)
    s1 = s2 * (1.0 / jnp.maximum(n1, 1e-12))
    zn = zn_ref[...]
    simb = jax.lax.dot_general(zn, s1, (((1,), (1,)), ((), ())),
                               preferred_element_type=jnp.float32)
    vrow = v_ref[...]
    sim_ref[...] = jnp.where(vrow > 0, simb, -jnp.inf)
    crow = c_ref[...]
    sub = jax.lax.broadcasted_iota(jnp.int32, (_CP, _TS), 0).astype(jnp.float32)
    M = jnp.where((sub == crow) & (vrow > 0), 1.0, 0.0)
    wpart = jax.lax.dot_general(M, s2, (((1,), (0,)), ((), ())),
                                preferred_element_type=jnp.float32)

    @pl.when(i == 0)
    def _():
        wt_ref[...] = wpart

    @pl.when(i > 0)
    def _():
        wt_ref[...] += wpart


def _top3_k(sim_ref, tv_ref, ti_ref, vals, idxs):
    i = pl.program_id(0)

    @pl.when(i == 0)
    def _():
        vals[...] = jnp.full((_B, 128), -jnp.inf, jnp.float32)
        idxs[...] = jnp.zeros((_B, 128), jnp.float32)

    x = sim_ref[...]
    lane = jax.lax.broadcasted_iota(jnp.int32, (_B, _TK), 1)
    for _ in range(3):
        m = jnp.max(x, axis=1, keepdims=True)
        pos = jnp.min(jnp.where(x == m, lane, _NP), axis=1, keepdims=True)
        gidx = (pos + i * _TK).astype(jnp.float32)
        x = jnp.where(lane == pos, -jnp.inf, x)
        v0 = vals[:, 0:1]; v1 = vals[:, 1:2]; v2 = vals[:, 2:3]
        i0 = idxs[:, 0:1]; i1 = idxs[:, 1:2]; i2 = idxs[:, 2:3]
        b0 = m > v0
        b1 = m > v1
        b2 = m > v2
        vals[:, 0:1] = jnp.where(b0, m, v0)
        idxs[:, 0:1] = jnp.where(b0, gidx, i0)
        vals[:, 1:2] = jnp.where(b0, v0, jnp.where(b1, m, v1))
        idxs[:, 1:2] = jnp.where(b0, i0, jnp.where(b1, gidx, i1))
        vals[:, 2:3] = jnp.where(b0 | b1, v1, jnp.where(b2, m, v2))
        idxs[:, 2:3] = jnp.where(b0 | b1, i1, jnp.where(b2, gidx, i2))

    @pl.when(i == (_NP // _TK) - 1)
    def _():
        tv_ref[...] = vals[...]
        ti_ref[...] = idxs[...]


def _loss_k(wt_ref, zn_ref, sc_ref, p_ref, tv_ref, sn0_ref, sn1_ref, sn2_ref,
            out_ref):
    lane = jax.lax.broadcasted_iota(jnp.int32, (_B, _CP), 1)
    maskc = lane < _C
    wt = wt_ref[...]
    n1 = jnp.sum(jnp.abs(wt), axis=1, keepdims=True)
    wn = wt * (1.0 / jnp.maximum(n1, 1e-12))
    zn = zn_ref[...]
    dist = jax.lax.dot_general(zn, wn, (((1,), (1,)), ((), ())),
                               preferred_element_type=jnp.float32)
    ld = jnp.where(maskc, dist, _NEG)
    md = jnp.max(ld, axis=1, keepdims=True)
    ed = jnp.exp(ld - md)
    sd = jnp.sum(ed, axis=1, keepdims=True)
    tgt = ed / sd
    lsm_d = ld - md - jnp.log(sd)
    sc = sc_ref[...]
    ls = jnp.where(maskc, sc, _NEG)
    ms = jnp.max(ls, axis=1, keepdims=True)
    es = jnp.exp(ls - ms)
    ss = jnp.sum(es, axis=1, keepdims=True)
    lsm_s = ls - ms - jnp.log(ss)
    kl = jnp.where(maskc, tgt * (lsm_d - lsm_s), 0.0)
    loss1 = jnp.sum(jnp.sum(kl, axis=1, keepdims=True), axis=0,
                    keepdims=True) / _B
    p = p_ref[...]
    tv = tv_ref[...]
    acc = jnp.zeros((_B, 1), jnp.float32)
    for k, sn_ref in enumerate((sn0_ref, sn1_ref, sn2_ref)):
        sn = sn_ref[...]
        d = p - sn
        diff = jnp.sum(jnp.where(maskc, d * d, 0.0), axis=1, keepdims=True)
        acc = acc + (-tv[:, k:k + 1]) * diff
    loss2 = jnp.sum(acc, axis=0, keepdims=True) / (_B * 3)
    loss = loss1 + _LAM * loss2
    out_ref[...] = jnp.broadcast_to(loss, (1, 128))


def kernel(z, W, supports_bank, labels_bank, ent_bank, scores_bank):
    f32 = jnp.float32
    Wp = jnp.pad(W, ((0, _CP - _C), (0, 0)))

    p, scores, ent_b, cls_b, zn = pl.pallas_call(
        _head_k,
        out_shape=[
            jax.ShapeDtypeStruct((_B, _CP), f32),
            jax.ShapeDtypeStruct((_B, _CP), f32),
            jax.ShapeDtypeStruct((_B, 1), f32),
            jax.ShapeDtypeStruct((_B, 1), f32),
            jax.ShapeDtypeStruct((_B, _D), f32),
        ],
    )(z, Wp)

    lb = jnp.pad(labels_bank, ((0, _NP - _NBANK), (0, _CP - _C)))
    cls_bank = pl.pallas_call(
        _cls_k,
        grid=(_NP // _TS,),
        in_specs=[pl.BlockSpec((_TS, _CP), lambda i: (i, 0))],
        out_specs=pl.BlockSpec((_TS, 1), lambda i: (i, 0)),
        out_shape=jax.ShapeDtypeStruct((_NP, 1), f32),
    )(lb)

    npad = _NP - _N
    ent_col = jnp.concatenate(
        [ent_bank[:, None], ent_b, jnp.full((npad, 1), jnp.inf, f32)], axis=0)
    cls_col = jnp.concatenate(
        [cls_bank[:_NBANK], cls_b, jnp.full((npad, 1), -1.0, f32)], axis=0)
    ent_row = ent_col.reshape(1, _NP)
    cls_row = cls_col.reshape(1, _NP)

    valid_col = pl.pallas_call(
        _rank_k,
        grid=(_NP // _TS,),
        in_specs=[
            pl.BlockSpec((_TS, 1), lambda i: (i, 0)),
            pl.BlockSpec((_TS, 1), lambda i: (i, 0)),
            pl.BlockSpec((1, _NP), lambda i: (0, 0)),
            pl.BlockSpec((1, _NP), lambda i: (0, 0)),
        ],
        out_specs=pl.BlockSpec((_TS, 1), lambda i: (i, 0)),
        out_shape=jax.ShapeDtypeStruct((_NP, 1), f32),
    )(ent_col, cls_col, ent_row, cls_row)
    valid_row = valid_col.reshape(1, _NP)

    supports_all = jnp.concatenate(
        [supports_bank, z, jnp.zeros((npad, _D), f32)], axis=0)

    sim, wt = pl.pallas_call(
        _big_k,
        grid=(_NP // _TS,),
        in_specs=[
            pl.BlockSpec((_TS, _D), lambda i: (i, 0)),
            pl.BlockSpec((1, _TS), lambda i: (0, i)),
            pl.BlockSpec((1, _TS), lambda i: (0, i)),
            pl.BlockSpec((_B, _D), lambda i: (0, 0)),
        ],
        out_specs=[
            pl.BlockSpec((_B, _TS), lambda i: (0, i)),
            pl.BlockSpec((_CP, _D), lambda i: (0, 0)),
        ],
        out_shape=[
            jax.ShapeDtypeStruct((_B, _NP), f32),
            jax.ShapeDtypeStruct((_CP, _D), f32),
        ],
    )(supports_all, valid_row, cls_row, zn)

    tv, ti = pl.pallas_call(
        _top3_k,
        grid=(_NP // _TK,),
        in_specs=[pl.BlockSpec((_B, _TK), lambda i: (0, i))],
        out_specs=[
            pl.BlockSpec((_B, 128), lambda i: (0, 0)),
            pl.BlockSpec((_B, 128), lambda i: (0, 0)),
        ],
        out_shape=[
            jax.ShapeDtypeStruct((_B, 128), f32),
            jax.ShapeDtypeStruct((_B, 128), f32),
        ],
        scratch_shapes=[
            pltpu.VMEM((_B, 128), f32),
            pltpu.VMEM((_B, 128), f32),
        ],
    )(sim)

    idx_near = ti[:, :3].astype(jnp.int32)
    scores_allp = jnp.concatenate(
        [jnp.pad(scores_bank, ((0, 0), (0, _CP - _C))), scores], axis=0)
    sn = jnp.take(scores_allp, idx_near.reshape(-1), axis=0)
    sn = sn.reshape(_B, 3, _CP)

    out = pl.pallas_call(
        _loss_k,
        out_shape=jax.ShapeDtypeStruct((1, 128), f32),
    )(wt, zn, scores, p, tv, sn[:, 0, :], sn[:, 1, :], sn[:, 2, :])

    return p[:, :_C], out[0, 0]


# speedup vs baseline: 4.7740x; 1.1241x over previous
import jax
import jax.numpy as jnp
from jax.experimental import pallas as pl
from jax.experimental.pallas import tpu as pltpu

_B, _D, _C, _NBANK, _FK, _LAM = 256, 2048, 200, 20000, 100, 0.1
_N = _NBANK + _B
_NP = 20480
_CP = 256
_TS = 512
_TK = 2048
_NEG = -1e30


def _head_k(z_ref, w_ref, p_ref, sc_ref, ent_ref, cls_ref, zn_ref):
    z = z_ref[...]
    w = w_ref[...]
    p = jax.lax.dot_general(z, w, (((1,), (1,)), ((), ())),
                            preferred_element_type=jnp.float32)
    lane = jax.lax.broadcasted_iota(jnp.int32, (_B, _CP), 1)
    maskc = lane < _C
    logits = jnp.where(maskc, p, _NEG)
    m = jnp.max(logits, axis=1, keepdims=True)
    e = jnp.exp(logits - m)
    s = jnp.sum(e, axis=1, keepdims=True)
    sc = e / s
    lsm = logits - m - jnp.log(s)
    ent = -jnp.sum(jnp.where(maskc, sc * lsm, 0.0), axis=1, keepdims=True)
    pos = jnp.min(jnp.where(logits == m, lane, _CP), axis=1, keepdims=True)
    n1 = jnp.sum(jnp.abs(z), axis=1, keepdims=True)
    p_ref[...] = p
    sc_ref[...] = sc
    ent_ref[...] = ent
    cls_ref[...] = pos.astype(jnp.float32)
    zn_ref[...] = z / jnp.maximum(n1, 1e-12)


def _cls_k(l_ref, o_ref):
    lab = l_ref[...]
    lane = jax.lax.broadcasted_iota(jnp.int32, lab.shape, 1).astype(jnp.float32)
    o_ref[...] = jnp.sum(lab * lane, axis=1, keepdims=True)


def _rank_k(ec_ref, cc_ref, er_ref, cr_ref, v_ref):
    i = pl.program_id(0)
    ej = ec_ref[...]
    cj = cc_ref[...]
    idxj = jax.lax.broadcasted_iota(jnp.int32, (_TS, 1), 0) + i * _TS
    cnt = jnp.zeros((_TS, 1), jnp.float32)
    for k0 in range(0, _NP, _TK):
        ek = er_ref[:, k0:k0 + _TK]
        ck = cr_ref[:, k0:k0 + _TK]
        idxk = jax.lax.broadcasted_iota(jnp.int32, (1, _TK), 1) + k0
        same = ck == cj
        less = (ek < ej) | ((ek == ej) & (idxk < idxj))
        cnt = cnt + jnp.sum(jnp.where(same & less, 1.0, 0.0),
                            axis=1, keepdims=True)
    v = (cnt < _FK) & (idxj < _N)
    v_ref[...] = v.astype(jnp.float32)


def _big_k(s_ref, t_ref, v_ref, c_ref, zn_ref, sim_ref, wt_ref):
    i = pl.program_id(0)
    S = jnp.where(i == (_NP // _TS) - 1, t_ref[...], s_ref[...])
    n2 = jnp.sum(S * S, axis=1, keepdims=True)
    s2 = S * (1.0 / jnp.maximum(jnp.sqrt(n2), 1e-12))
    n1 = jnp.sum(jnp.abs(s2), axis=1, keepdims=True)
    s1 = s2 * (1.0 / jnp.maximum(n1, 1e-12))
    zn = zn_ref[...]
    simb = jax.lax.dot_general(zn, s1, (((1,), (1,)), ((), ())),
                               preferred_element_type=jnp.float32)
    vrow = v_ref[...]
    sim_ref[...] = jnp.where(vrow > 0, simb, -jnp.inf)
    crow = c_ref[...]
    sub = jax.lax.broadcasted_iota(jnp.int32, (_CP, _TS), 0).astype(jnp.float32)
    M = jnp.where((sub == crow) & (vrow > 0), 1.0, 0.0)
    wpart = jax.lax.dot_general(M, s2, (((1,), (0,)), ((), ())),
                                preferred_element_type=jnp.float32)

    @pl.when(i == 0)
    def _():
        wt_ref[...] = wpart

    @pl.when(i > 0)
    def _():
        wt_ref[...] += wpart


def _top3_k(sim_ref, tv_ref, ti_ref, vals, idxs):
    i = pl.program_id(0)

    @pl.when(i == 0)
    def _():
        vals[...] = jnp.full((_B, 128), -jnp.inf, jnp.float32)
        idxs[...] = jnp.zeros((_B, 128), jnp.float32)

    x = sim_ref[...]
    lane = jax.lax.broadcasted_iota(jnp.int32, (_B, _TK), 1)
    for _ in range(3):
        m = jnp.max(x, axis=1, keepdims=True)
        pos = jnp.min(jnp.where(x == m, lane, _NP), axis=1, keepdims=True)
        gidx = (pos + i * _TK).astype(jnp.float32)
        x = jnp.where(lane == pos, -jnp.inf, x)
        v0 = vals[:, 0:1]; v1 = vals[:, 1:2]; v2 = vals[:, 2:3]
        i0 = idxs[:, 0:1]; i1 = idxs[:, 1:2]; i2 = idxs[:, 2:3]
        b0 = m > v0
        b1 = m > v1
        b2 = m > v2
        vals[:, 0:1] = jnp.where(b0, m, v0)
        idxs[:, 0:1] = jnp.where(b0, gidx, i0)
        vals[:, 1:2] = jnp.where(b0, v0, jnp.where(b1, m, v1))
        idxs[:, 1:2] = jnp.where(b0, i0, jnp.where(b1, gidx, i1))
        vals[:, 2:3] = jnp.where(b0 | b1, v1, jnp.where(b2, m, v2))
        idxs[:, 2:3] = jnp.where(b0 | b1, i1, jnp.where(b2, gidx, i2))

    @pl.when(i == (_NP // _TK) - 1)
    def _():
        tv_ref[...] = vals[...]
        ti_ref[...] = idxs[...]


def _loss_k(wt_ref, zn_ref, sc_ref, p_ref, tv_ref, sn0_ref, sn1_ref, sn2_ref,
            out_ref):
    lane = jax.lax.broadcasted_iota(jnp.int32, (_B, _CP), 1)
    maskc = lane < _C
    wt = wt_ref[...]
    n1 = jnp.sum(jnp.abs(wt), axis=1, keepdims=True)
    wn = wt * (1.0 / jnp.maximum(n1, 1e-12))
    zn = zn_ref[...]
    dist = jax.lax.dot_general(zn, wn, (((1,), (1,)), ((), ())),
                               preferred_element_type=jnp.float32)
    ld = jnp.where(maskc, dist, _NEG)
    md = jnp.max(ld, axis=1, keepdims=True)
    ed = jnp.exp(ld - md)
    sd = jnp.sum(ed, axis=1, keepdims=True)
    tgt = ed / sd
    lsm_d = ld - md - jnp.log(sd)
    sc = sc_ref[...]
    ls = jnp.where(maskc, sc, _NEG)
    ms = jnp.max(ls, axis=1, keepdims=True)
    es = jnp.exp(ls - ms)
    ss = jnp.sum(es, axis=1, keepdims=True)
    lsm_s = ls - ms - jnp.log(ss)
    kl = jnp.where(maskc, tgt * (lsm_d - lsm_s), 0.0)
    loss1 = jnp.sum(jnp.sum(kl, axis=1, keepdims=True), axis=0,
                    keepdims=True) / _B
    p = p_ref[...]
    tv = tv_ref[...]
    acc = jnp.zeros((_B, 1), jnp.float32)
    for k, sn_ref in enumerate((sn0_ref, sn1_ref, sn2_ref)):
        sn = sn_ref[...]
        d = p - sn
        diff = jnp.sum(jnp.where(maskc, d * d, 0.0), axis=1, keepdims=True)
        acc = acc + (-tv[:, k:k + 1]) * diff
    loss2 = jnp.sum(acc, axis=0, keepdims=True) / (_B * 3)
    loss = loss1 + _LAM * loss2
    out_ref[...] = jnp.broadcast_to(loss, (1, 128))


def kernel(z, W, supports_bank, labels_bank, ent_bank, scores_bank):
    f32 = jnp.float32
    Wp = jnp.pad(W, ((0, _CP - _C), (0, 0)))

    p, scores, ent_b, cls_b, zn = pl.pallas_call(
        _head_k,
        out_shape=[
            jax.ShapeDtypeStruct((_B, _CP), f32),
            jax.ShapeDtypeStruct((_B, _CP), f32),
            jax.ShapeDtypeStruct((_B, 1), f32),
            jax.ShapeDtypeStruct((_B, 1), f32),
            jax.ShapeDtypeStruct((_B, _D), f32),
        ],
    )(z, Wp)

    cls_bank = pl.pallas_call(
        _cls_k,
        grid=(_NP // _TS,),
        in_specs=[pl.BlockSpec((_TS, _C), lambda i: (i, 0))],
        out_specs=pl.BlockSpec((_TS, 1), lambda i: (i, 0)),
        out_shape=jax.ShapeDtypeStruct((_NBANK, 1), f32),
    )(labels_bank)

    npad = _NP - _N
    ent_col = jnp.concatenate(
        [ent_bank[:, None], ent_b, jnp.full((npad, 1), jnp.inf, f32)], axis=0)
    cls_col = jnp.concatenate(
        [cls_bank, cls_b, jnp.full((npad, 1), -1.0, f32)], axis=0)
    ent_row = ent_col.reshape(1, _NP)
    cls_row = cls_col.reshape(1, _NP)

    valid_col = pl.pallas_call(
        _rank_k,
        grid=(_NP // _TS,),
        in_specs=[
            pl.BlockSpec((_TS, 1), lambda i: (i, 0)),
            pl.BlockSpec((_TS, 1), lambda i: (i, 0)),
            pl.BlockSpec((1, _NP), lambda i: (0, 0)),
            pl.BlockSpec((1, _NP), lambda i: (0, 0)),
        ],
        out_specs=pl.BlockSpec((_TS, 1), lambda i: (i, 0)),
        out_shape=jax.ShapeDtypeStruct((_NP, 1), f32),
    )(ent_col, cls_col, ent_row, cls_row)
    valid_row = valid_col.reshape(1, _NP)

    ntiles = _NP // _TS
    tail_bank = _TS * (ntiles - 1)
    tail = jnp.concatenate(
        [supports_bank[tail_bank:], z,
         jnp.zeros((_TS - (_NBANK - tail_bank) - _B, _D), f32)], axis=0)

    sim, wt = pl.pallas_call(
        _big_k,
        grid=(ntiles,),
        in_specs=[
            pl.BlockSpec((_TS, _D), lambda i: (jnp.minimum(i, ntiles - 2), 0)),
            pl.BlockSpec((_TS, _D), lambda i: (0, 0)),
            pl.BlockSpec((1, _TS), lambda i: (0, i)),
            pl.BlockSpec((1, _TS), lambda i: (0, i)),
            pl.BlockSpec((_B, _D), lambda i: (0, 0)),
        ],
        out_specs=[
            pl.BlockSpec((_B, _TS), lambda i: (0, i)),
            pl.BlockSpec((_CP, _D), lambda i: (0, 0)),
        ],
        out_shape=[
            jax.ShapeDtypeStruct((_B, _NP), f32),
            jax.ShapeDtypeStruct((_CP, _D), f32),
        ],
    )(supports_bank, tail, valid_row, cls_row, zn)

    tv, ti = pl.pallas_call(
        _top3_k,
        grid=(_NP // _TK,),
        in_specs=[pl.BlockSpec((_B, _TK), lambda i: (0, i))],
        out_specs=[
            pl.BlockSpec((_B, 128), lambda i: (0, 0)),
            pl.BlockSpec((_B, 128), lambda i: (0, 0)),
        ],
        out_shape=[
            jax.ShapeDtypeStruct((_B, 128), f32),
            jax.ShapeDtypeStruct((_B, 128), f32),
        ],
        scratch_shapes=[
            pltpu.VMEM((_B, 128), f32),
            pltpu.VMEM((_B, 128), f32),
        ],
    )(sim)

    idx_near = ti[:, :3].astype(jnp.int32)
    scores_allp = jnp.concatenate(
        [jnp.pad(scores_bank, ((0, 0), (0, _CP - _C))), scores], axis=0)
    sn = jnp.take(scores_allp, idx_near.reshape(-1), axis=0)
    sn = sn.reshape(_B, 3, _CP)

    out = pl.pallas_call(
        _loss_k,
        out_shape=jax.ShapeDtypeStruct((1, 128), f32),
    )(wt, zn, scores, p, tv, sn[:, 0, :], sn[:, 1, :], sn[:, 2, :])

    return p[:, :_C], out[0, 0]


# speedup vs baseline: 5.9974x; 1.2563x over previous
import jax
import jax.numpy as jnp
from jax.experimental import pallas as pl
from jax.experimental.pallas import tpu as pltpu

_B, _D, _C, _NBANK, _FK, _LAM = 256, 2048, 200, 20000, 100, 0.1
_N = _NBANK + _B
_NP = 20480
_CP = 256
_TS = 512
_TK = 2048
_NEG = -1e30


def _head_k(z_ref, w_ref, p_ref, sc_ref, ent_ref, cls_ref, zn_ref):
    z = z_ref[...]
    w = w_ref[...]
    p = jax.lax.dot_general(z, w, (((1,), (1,)), ((), ())),
                            preferred_element_type=jnp.float32)
    lane = jax.lax.broadcasted_iota(jnp.int32, (_B, _CP), 1)
    maskc = lane < _C
    logits = jnp.where(maskc, p, _NEG)
    m = jnp.max(logits, axis=1, keepdims=True)
    e = jnp.exp(logits - m)
    s = jnp.sum(e, axis=1, keepdims=True)
    sc = e / s
    lsm = logits - m - jnp.log(s)
    ent = -jnp.sum(jnp.where(maskc, sc * lsm, 0.0), axis=1, keepdims=True)
    pos = jnp.min(jnp.where(logits == m, lane, _CP), axis=1, keepdims=True)
    n1 = jnp.sum(jnp.abs(z), axis=1, keepdims=True)
    p_ref[...] = p
    sc_ref[...] = sc
    ent_ref[...] = ent
    cls_ref[...] = pos.astype(jnp.float32)
    zn_ref[...] = z / jnp.maximum(n1, 1e-12)


def _cls_k(l_ref, o_ref):
    lab = l_ref[...]
    lane = jax.lax.broadcasted_iota(jnp.int32, lab.shape, 1).astype(jnp.float32)
    o_ref[...] = jnp.sum(lab * lane, axis=1, keepdims=True)


def _rank_k(ec_ref, cc_ref, er_ref, cr_ref, ed_ref, cd_ref, v_ref):
    i = pl.program_id(0)
    ej = ec_ref[...]
    cj = cc_ref[...]
    cnt = jnp.zeros((_TS, 1), jnp.int32)
    for k in range(_NP // _TS):
        ek = er_ref[:, k * _TS:(k + 1) * _TS]
        ck = cr_ref[:, k * _TS:(k + 1) * _TS]
        thr = ej + jnp.where(i > k, 1, 0)
        mask = (ck == cj) & (ek < thr)
        cnt = cnt + jnp.sum(mask.astype(jnp.int32), axis=1, keepdims=True)
    ed = ed_ref[...]
    cd = cd_ref[...]
    li = jax.lax.broadcasted_iota(jnp.int32, (_TS, _TS), 1)
    si = jax.lax.broadcasted_iota(jnp.int32, (_TS, _TS), 0)
    tie = (cd == cj) & (ed == ej) & (li < si)
    cnt = cnt + jnp.sum(tie.astype(jnp.int32), axis=1, keepdims=True)
    idxj = jax.lax.broadcasted_iota(jnp.int32, (_TS, 1), 0) + i * _TS
    v = (cnt < _FK) & (idxj < _N)
    v_ref[...] = v.astype(jnp.float32)


def _big_k(s_ref, t_ref, v_ref, c_ref, zn_ref, sim_ref, wt_ref):
    i = pl.program_id(0)
    S = jnp.where(i == (_NP // _TS) - 1, t_ref[...], s_ref[...])
    n2 = jnp.sum(S * S, axis=1, keepdims=True)
    s2 = S * (1.0 / jnp.maximum(jnp.sqrt(n2), 1e-12))
    n1 = jnp.sum(jnp.abs(s2), axis=1, keepdims=True)
    s1 = s2 * (1.0 / jnp.maximum(n1, 1e-12))
    zn = zn_ref[...]
    simb = jax.lax.dot_general(zn, s1, (((1,), (1,)), ((), ())),
                               preferred_element_type=jnp.float32)
    vrow = v_ref[...]
    sim_ref[...] = jnp.where(vrow > 0, simb, -jnp.inf)
    crow = c_ref[...]
    sub = jax.lax.broadcasted_iota(jnp.int32, (_CP, _TS), 0).astype(jnp.float32)
    M = jnp.where((sub == crow) & (vrow > 0), 1.0, 0.0)
    wpart = jax.lax.dot_general(M, s2, (((1,), (0,)), ((), ())),
                                preferred_element_type=jnp.float32)

    @pl.when(i == 0)
    def _():
        wt_ref[...] = wpart

    @pl.when(i > 0)
    def _():
        wt_ref[...] += wpart


def _top3_k(sim_ref, tv_ref, ti_ref, vals, idxs):
    i = pl.program_id(0)

    @pl.when(i == 0)
    def _():
        vals[...] = jnp.full((_B, 128), -jnp.inf, jnp.float32)
        idxs[...] = jnp.zeros((_B, 128), jnp.float32)

    x = sim_ref[...]
    lane = jax.lax.broadcasted_iota(jnp.int32, (_B, _TK), 1)
    for _ in range(3):
        m = jnp.max(x, axis=1, keepdims=True)
        pos = jnp.min(jnp.where(x == m, lane, _NP), axis=1, keepdims=True)
        gidx = (pos + i * _TK).astype(jnp.float32)
        x = jnp.where(lane == pos, -jnp.inf, x)
        v0 = vals[:, 0:1]; v1 = vals[:, 1:2]; v2 = vals[:, 2:3]
        i0 = idxs[:, 0:1]; i1 = idxs[:, 1:2]; i2 = idxs[:, 2:3]
        b0 = m > v0
        b1 = m > v1
        b2 = m > v2
        vals[:, 0:1] = jnp.where(b0, m, v0)
        idxs[:, 0:1] = jnp.where(b0, gidx, i0)
        vals[:, 1:2] = jnp.where(b0, v0, jnp.where(b1, m, v1))
        idxs[:, 1:2] = jnp.where(b0, i0, jnp.where(b1, gidx, i1))
        vals[:, 2:3] = jnp.where(b0 | b1, v1, jnp.where(b2, m, v2))
        idxs[:, 2:3] = jnp.where(b0 | b1, i1, jnp.where(b2, gidx, i2))

    @pl.when(i == (_NP // _TK) - 1)
    def _():
        tv_ref[...] = vals[...]
        ti_ref[...] = idxs[...]


def _loss_k(wt_ref, zn_ref, sc_ref, p_ref, tv_ref, sn0_ref, sn1_ref, sn2_ref,
            out_ref):
    lane = jax.lax.broadcasted_iota(jnp.int32, (_B, _CP), 1)
    maskc = lane < _C
    wt = wt_ref[...]
    n1 = jnp.sum(jnp.abs(wt), axis=1, keepdims=True)
    wn = wt * (1.0 / jnp.maximum(n1, 1e-12))
    zn = zn_ref[...]
    dist = jax.lax.dot_general(zn, wn, (((1,), (1,)), ((), ())),
                               preferred_element_type=jnp.float32)
    ld = jnp.where(maskc, dist, _NEG)
    md = jnp.max(ld, axis=1, keepdims=True)
    ed = jnp.exp(ld - md)
    sd = jnp.sum(ed, axis=1, keepdims=True)
    tgt = ed / sd
    lsm_d = ld - md - jnp.log(sd)
    sc = sc_ref[...]
    ls = jnp.where(maskc, sc, _NEG)
    ms = jnp.max(ls, axis=1, keepdims=True)
    es = jnp.exp(ls - ms)
    ss = jnp.sum(es, axis=1, keepdims=True)
    lsm_s = ls - ms - jnp.log(ss)
    kl = jnp.where(maskc, tgt * (lsm_d - lsm_s), 0.0)
    loss1 = jnp.sum(jnp.sum(kl, axis=1, keepdims=True), axis=0,
                    keepdims=True) / _B
    p = p_ref[...]
    tv = tv_ref[...]
    acc = jnp.zeros((_B, 1), jnp.float32)
    for k, sn_ref in enumerate((sn0_ref, sn1_ref, sn2_ref)):
        sn = sn_ref[...]
        d = p - sn
        diff = jnp.sum(jnp.where(maskc, d * d, 0.0), axis=1, keepdims=True)
        acc = acc + (-tv[:, k:k + 1]) * diff
    loss2 = jnp.sum(acc, axis=0, keepdims=True) / (_B * 3)
    loss = loss1 + _LAM * loss2
    out_ref[...] = jnp.broadcast_to(loss, (1, 128))


def kernel(z, W, supports_bank, labels_bank, ent_bank, scores_bank):
    f32 = jnp.float32
    Wp = jnp.pad(W, ((0, _CP - _C), (0, 0)))

    p, scores, ent_b, cls_b, zn = pl.pallas_call(
        _head_k,
        out_shape=[
            jax.ShapeDtypeStruct((_B, _CP), f32),
            jax.ShapeDtypeStruct((_B, _CP), f32),
            jax.ShapeDtypeStruct((_B, 1), f32),
            jax.ShapeDtypeStruct((_B, 1), f32),
            jax.ShapeDtypeStruct((_B, _D), f32),
        ],
    )(z, Wp)

    cls_bank = pl.pallas_call(
        _cls_k,
        grid=(_NP // _TS,),
        in_specs=[pl.BlockSpec((_TS, _C), lambda i: (i, 0))],
        out_specs=pl.BlockSpec((_TS, 1), lambda i: (i, 0)),
        out_shape=jax.ShapeDtypeStruct((_NBANK, 1), f32),
    )(labels_bank)

    npad = _NP - _N
    ent_col = jnp.concatenate(
        [ent_bank[:, None], ent_b, jnp.full((npad, 1), jnp.inf, f32)], axis=0)
    cls_col = jnp.concatenate(
        [cls_bank, cls_b, jnp.full((npad, 1), -1.0, f32)], axis=0)
    key_col = jax.lax.bitcast_convert_type(jnp.abs(ent_col), jnp.int32)
    key_row = key_col.reshape(1, _NP)
    cls_row = cls_col.reshape(1, _NP)

    valid_col = pl.pallas_call(
        _rank_k,
        grid=(_NP // _TS,),
        in_specs=[
            pl.BlockSpec((_TS, 1), lambda i: (i, 0)),
            pl.BlockSpec((_TS, 1), lambda i: (i, 0)),
            pl.BlockSpec((1, _NP), lambda i: (0, 0)),
            pl.BlockSpec((1, _NP), lambda i: (0, 0)),
            pl.BlockSpec((1, _TS), lambda i: (0, i)),
            pl.BlockSpec((1, _TS), lambda i: (0, i)),
        ],
        out_specs=pl.BlockSpec((_TS, 1), lambda i: (i, 0)),
        out_shape=jax.ShapeDtypeStruct((_NP, 1), f32),
    )(key_col, cls_col, key_row, cls_row, key_row, cls_row)
    valid_row = valid_col.reshape(1, _NP)

    ntiles = _NP // _TS
    tail_bank = _TS * (ntiles - 1)
    tail = jnp.concatenate(
        [supports_bank[tail_bank:], z,
         jnp.zeros((_TS - (_NBANK - tail_bank) - _B, _D), f32)], axis=0)

    sim, wt = pl.pallas_call(
        _big_k,
        grid=(ntiles,),
        in_specs=[
            pl.BlockSpec((_TS, _D), lambda i: (jnp.minimum(i, ntiles - 2), 0)),
            pl.BlockSpec((_TS, _D), lambda i: (0, 0)),
            pl.BlockSpec((1, _TS), lambda i: (0, i)),
            pl.BlockSpec((1, _TS), lambda i: (0, i)),
            pl.BlockSpec((_B, _D), lambda i: (0, 0)),
        ],
        out_specs=[
            pl.BlockSpec((_B, _TS), lambda i: (0, i)),
            pl.BlockSpec((_CP, _D), lambda i: (0, 0)),
        ],
        out_shape=[
            jax.ShapeDtypeStruct((_B, _NP), f32),
            jax.ShapeDtypeStruct((_CP, _D), f32),
        ],
    )(supports_bank, tail, valid_row, cls_row, zn)

    tv, ti = pl.pallas_call(
        _top3_k,
        grid=(_NP // _TK,),
        in_specs=[pl.BlockSpec((_B, _TK), lambda i: (0, i))],
        out_specs=[
            pl.BlockSpec((_B, 128), lambda i: (0, 0)),
            pl.BlockSpec((_B, 128), lambda i: (0, 0)),
        ],
        out_shape=[
            jax.ShapeDtypeStruct((_B, 128), f32),
            jax.ShapeDtypeStruct((_B, 128), f32),
        ],
        scratch_shapes=[
            pltpu.VMEM((_B, 128), f32),
            pltpu.VMEM((_B, 128), f32),
        ],
    )(sim)

    idx_near = ti[:, :3].astype(jnp.int32)
    scores_allp = jnp.concatenate(
        [jnp.pad(scores_bank, ((0, 0), (0, _CP - _C))), scores], axis=0)
    sn = jnp.take(scores_allp, idx_near.reshape(-1), axis=0)
    sn = sn.reshape(_B, 3, _CP)

    out = pl.pallas_call(
        _loss_k,
        out_shape=jax.ShapeDtypeStruct((1, 128), f32),
    )(wt, zn, scores, p, tv, sn[:, 0, :], sn[:, 1, :], sn[:, 2, :])

    return p[:, :_C], out[0, 0]


# speedup vs baseline: 6.3515x; 1.0591x over previous
import jax
import jax.numpy as jnp
from jax.experimental import pallas as pl
from jax.experimental.pallas import tpu as pltpu

_B, _D, _C, _NBANK, _FK, _LAM = 256, 2048, 200, 20000, 100, 0.1
_N = _NBANK + _B
_NP = 20480
_CP = 256
_TS = 512
_TK = 2048
_NEG = -1e30


def _head_k(z_ref, w_ref, p_ref, sc_ref, ent_ref, cls_ref, zn_ref):
    z = z_ref[...]
    w = w_ref[...]
    p = jax.lax.dot_general(z, w, (((1,), (1,)), ((), ())),
                            preferred_element_type=jnp.float32)
    lane = jax.lax.broadcasted_iota(jnp.int32, (_B, _CP), 1)
    maskc = lane < _C
    logits = jnp.where(maskc, p, _NEG)
    m = jnp.max(logits, axis=1, keepdims=True)
    e = jnp.exp(logits - m)
    s = jnp.sum(e, axis=1, keepdims=True)
    sc = e / s
    lsm = logits - m - jnp.log(s)
    ent = -jnp.sum(jnp.where(maskc, sc * lsm, 0.0), axis=1, keepdims=True)
    pos = jnp.min(jnp.where(logits == m, lane, _CP), axis=1, keepdims=True)
    n1 = jnp.sum(jnp.abs(z), axis=1, keepdims=True)
    p_ref[...] = p
    sc_ref[...] = sc
    ent_ref[...] = ent
    cls_ref[...] = pos.astype(jnp.float32)
    zn_ref[...] = z / jnp.maximum(n1, 1e-12)


def _cls_k(l_ref, o_ref):
    lab = l_ref[...]
    sub = jax.lax.broadcasted_iota(jnp.int32, lab.shape, 0).astype(jnp.float32)
    o_ref[...] = jnp.sum(lab * sub, axis=0, keepdims=True)


def _rank_k(ec_ref, cc_ref, er_ref, cr_ref, ed_ref, cd_ref, v_ref):
    i = pl.program_id(0)
    ej = ec_ref[...]
    cj = cc_ref[...]
    cnt = jnp.zeros((_TS, 1), jnp.int32)
    for k in range(_NP // _TS):
        ek = er_ref[:, k * _TS:(k + 1) * _TS]
        ck = cr_ref[:, k * _TS:(k + 1) * _TS]
        thr = ej + jnp.where(i > k, 1, 0)
        mask = (ck == cj) & (ek < thr)
        cnt = cnt + jnp.sum(mask.astype(jnp.int32), axis=1, keepdims=True)
    ed = ed_ref[...]
    cd = cd_ref[...]
    li = jax.lax.broadcasted_iota(jnp.int32, (_TS, _TS), 1)
    si = jax.lax.broadcasted_iota(jnp.int32, (_TS, _TS), 0)
    tie = (cd == cj) & (ed == ej) & (li < si)
    cnt = cnt + jnp.sum(tie.astype(jnp.int32), axis=1, keepdims=True)
    idxj = jax.lax.broadcasted_iota(jnp.int32, (_TS, 1), 0) + i * _TS
    v = (cnt < _FK) & (idxj < _N)
    v_ref[...] = v.astype(jnp.float32)


def _big_k(s_ref, t_ref, v_ref, c_ref, zn_ref, sim_ref, wt_ref):
    i = pl.program_id(0)
    S = jnp.where(i == (_NP // _TS) - 1, t_ref[...], s_ref[...])
    n2 = jnp.sum(S * S, axis=1, keepdims=True)
    s2 = S * (1.0 / jnp.maximum(jnp.sqrt(n2), 1e-12))
    n1 = jnp.sum(jnp.abs(s2), axis=1, keepdims=True)
    s1 = s2 * (1.0 / jnp.maximum(n1, 1e-12))
    zn = zn_ref[...]
    simb = jax.lax.dot_general(zn, s1, (((1,), (1,)), ((), ())),
                               preferred_element_type=jnp.float32)
    vrow = v_ref[...]
    sim_ref[...] = jnp.where(vrow > 0, simb, -jnp.inf)
    crow = c_ref[...]
    sub = jax.lax.broadcasted_iota(jnp.int32, (_CP, _TS), 0).astype(jnp.float32)
    M = jnp.where((sub == crow) & (vrow > 0), 1.0, 0.0)
    wpart = jax.lax.dot_general(M, s2, (((1,), (0,)), ((), ())),
                                preferred_element_type=jnp.float32)

    @pl.when(i == 0)
    def _():
        wt_ref[...] = wpart

    @pl.when(i > 0)
    def _():
        wt_ref[...] += wpart


def _top3_k(sim_ref, tv_ref, ti_ref, vals, idxs):
    i = pl.program_id(0)

    @pl.when(i == 0)
    def _():
        vals[...] = jnp.full((_B, 128), -jnp.inf, jnp.float32)
        idxs[...] = jnp.zeros((_B, 128), jnp.float32)

    x = sim_ref[...]
    lane = jax.lax.broadcasted_iota(jnp.int32, (_B, _TK), 1)
    for _ in range(3):
        m = jnp.max(x, axis=1, keepdims=True)
        pos = jnp.min(jnp.where(x == m, lane, _NP), axis=1, keepdims=True)
        gidx = (pos + i * _TK).astype(jnp.float32)
        x = jnp.where(lane == pos, -jnp.inf, x)
        v0 = vals[:, 0:1]; v1 = vals[:, 1:2]; v2 = vals[:, 2:3]
        i0 = idxs[:, 0:1]; i1 = idxs[:, 1:2]; i2 = idxs[:, 2:3]
        b0 = m > v0
        b1 = m > v1
        b2 = m > v2
        vals[:, 0:1] = jnp.where(b0, m, v0)
        idxs[:, 0:1] = jnp.where(b0, gidx, i0)
        vals[:, 1:2] = jnp.where(b0, v0, jnp.where(b1, m, v1))
        idxs[:, 1:2] = jnp.where(b0, i0, jnp.where(b1, gidx, i1))
        vals[:, 2:3] = jnp.where(b0 | b1, v1, jnp.where(b2, m, v2))
        idxs[:, 2:3] = jnp.where(b0 | b1, i1, jnp.where(b2, gidx, i2))

    @pl.when(i == (_NP // _TK) - 1)
    def _():
        tv_ref[...] = vals[...]
        ti_ref[...] = idxs[...]


def _loss_k(wt_ref, zn_ref, sc_ref, p_ref, tv_ref, sn0_ref, sn1_ref, sn2_ref,
            out_ref):
    lane = jax.lax.broadcasted_iota(jnp.int32, (_B, _CP), 1)
    maskc = lane < _C
    wt = wt_ref[...]
    n1 = jnp.sum(jnp.abs(wt), axis=1, keepdims=True)
    wn = wt * (1.0 / jnp.maximum(n1, 1e-12))
    zn = zn_ref[...]
    dist = jax.lax.dot_general(zn, wn, (((1,), (1,)), ((), ())),
                               preferred_element_type=jnp.float32)
    ld = jnp.where(maskc, dist, _NEG)
    md = jnp.max(ld, axis=1, keepdims=True)
    ed = jnp.exp(ld - md)
    sd = jnp.sum(ed, axis=1, keepdims=True)
    tgt = ed / sd
    lsm_d = ld - md - jnp.log(sd)
    sc = sc_ref[...]
    ls = jnp.where(maskc, sc, _NEG)
    ms = jnp.max(ls, axis=1, keepdims=True)
    es = jnp.exp(ls - ms)
    ss = jnp.sum(es, axis=1, keepdims=True)
    lsm_s = ls - ms - jnp.log(ss)
    kl = jnp.where(maskc, tgt * (lsm_d - lsm_s), 0.0)
    loss1 = jnp.sum(jnp.sum(kl, axis=1, keepdims=True), axis=0,
                    keepdims=True) / _B
    p = p_ref[...]
    tv = tv_ref[...]
    acc = jnp.zeros((_B, 1), jnp.float32)
    for k, sn_ref in enumerate((sn0_ref, sn1_ref, sn2_ref)):
        sn = sn_ref[...]
        d = p - sn
        diff = jnp.sum(jnp.where(maskc, d * d, 0.0), axis=1, keepdims=True)
        acc = acc + (-tv[:, k:k + 1]) * diff
    loss2 = jnp.sum(acc, axis=0, keepdims=True) / (_B * 3)
    loss = loss1 + _LAM * loss2
    out_ref[...] = jnp.broadcast_to(loss, (1, 128))


def kernel(z, W, supports_bank, labels_bank, ent_bank, scores_bank):
    f32 = jnp.float32
    Wp = jnp.pad(W, ((0, _CP - _C), (0, 0)))

    p, scores, ent_b, cls_b, zn = pl.pallas_call(
        _head_k,
        out_shape=[
            jax.ShapeDtypeStruct((_B, _CP), f32),
            jax.ShapeDtypeStruct((_B, _CP), f32),
            jax.ShapeDtypeStruct((_B, 1), f32),
            jax.ShapeDtypeStruct((_B, 1), f32),
            jax.ShapeDtypeStruct((_B, _D), f32),
        ],
    )(z, Wp)

    cls_bank = pl.pallas_call(
        _cls_k,
        grid=(10,),
        in_specs=[pl.BlockSpec((_C, 2048), lambda i: (0, i))],
        out_specs=pl.BlockSpec((1, 2048), lambda i: (0, i)),
        out_shape=jax.ShapeDtypeStruct((1, _NBANK), f32),
    )(labels_bank.T).reshape(_NBANK, 1)

    npad = _NP - _N
    ent_col = jnp.concatenate(
        [ent_bank[:, None], ent_b, jnp.full((npad, 1), jnp.inf, f32)], axis=0)
    cls_col = jnp.concatenate(
        [cls_bank, cls_b, jnp.full((npad, 1), -1.0, f32)], axis=0)
    key_col = jax.lax.bitcast_convert_type(jnp.abs(ent_col), jnp.int32)
    key_row = key_col.reshape(1, _NP)
    cls_row = cls_col.reshape(1, _NP)

    valid_col = pl.pallas_call(
        _rank_k,
        grid=(_NP // _TS,),
        in_specs=[
            pl.BlockSpec((_TS, 1), lambda i: (i, 0)),
            pl.BlockSpec((_TS, 1), lambda i: (i, 0)),
            pl.BlockSpec((1, _NP), lambda i: (0, 0)),
            pl.BlockSpec((1, _NP), lambda i: (0, 0)),
            pl.BlockSpec((1, _TS), lambda i: (0, i)),
            pl.BlockSpec((1, _TS), lambda i: (0, i)),
        ],
        out_specs=pl.BlockSpec((_TS, 1), lambda i: (i, 0)),
        out_shape=jax.ShapeDtypeStruct((_NP, 1), f32),
    )(key_col, cls_col, key_row, cls_row, key_row, cls_row)
    valid_row = valid_col.reshape(1, _NP)

    ntiles = _NP // _TS
    tail_bank = _TS * (ntiles - 1)
    tail = jnp.concatenate(
        [supports_bank[tail_bank:], z,
         jnp.zeros((_TS - (_NBANK - tail_bank) - _B, _D), f32)], axis=0)

    sim, wt = pl.pallas_call(
        _big_k,
        grid=(ntiles,),
        in_specs=[
            pl.BlockSpec((_TS, _D), lambda i: (jnp.minimum(i, ntiles - 2), 0)),
            pl.BlockSpec((_TS, _D), lambda i: (0, 0)),
            pl.BlockSpec((1, _TS), lambda i: (0, i)),
            pl.BlockSpec((1, _TS), lambda i: (0, i)),
            pl.BlockSpec((_B, _D), lambda i: (0, 0)),
        ],
        out_specs=[
            pl.BlockSpec((_B, _TS), lambda i: (0, i)),
            pl.BlockSpec((_CP, _D), lambda i: (0, 0)),
        ],
        out_shape=[
            jax.ShapeDtypeStruct((_B, _NP), f32),
            jax.ShapeDtypeStruct((_CP, _D), f32),
        ],
    )(supports_bank, tail, valid_row, cls_row, zn)

    tv, ti = pl.pallas_call(
        _top3_k,
        grid=(_NP // _TK,),
        in_specs=[pl.BlockSpec((_B, _TK), lambda i: (0, i))],
        out_specs=[
            pl.BlockSpec((_B, 128), lambda i: (0, 0)),
            pl.BlockSpec((_B, 128), lambda i: (0, 0)),
        ],
        out_shape=[
            jax.ShapeDtypeStruct((_B, 128), f32),
            jax.ShapeDtypeStruct((_B, 128), f32),
        ],
        scratch_shapes=[
            pltpu.VMEM((_B, 128), f32),
            pltpu.VMEM((_B, 128), f32),
        ],
    )(sim)

    idx_near = ti[:, :3].astype(jnp.int32)
    scores_allp = jnp.concatenate(
        [jnp.pad(scores_bank, ((0, 0), (0, _CP - _C))), scores], axis=0)
    sn = jnp.take(scores_allp, idx_near.reshape(-1), axis=0)
    sn = sn.reshape(_B, 3, _CP)

    out = pl.pallas_call(
        _loss_k,
        out_shape=jax.ShapeDtypeStruct((1, 128), f32),
    )(wt, zn, scores, p, tv, sn[:, 0, :], sn[:, 1, :], sn[:, 2, :])

    return p[:, :_C], out[0, 0]
